# baseline (speedup 1.0000x reference)
import jax
import jax.numpy as jnp
from jax import lax
from jax.experimental import pallas as pl
from jax.experimental.pallas import tpu as pltpu
from jax.experimental.pallas import tpu_sc as plsc

N = 50000
E = 800000
HID = 64
DM = 128
M = 8
D = 512
D0 = 16

NC = 2
NS = 16

CHUNK = 1280
NCHUNK = -(-N // CHUNK)
NPAD_BIG = NCHUNK * CHUNK
CPC = -(-NCHUNK // NC)
EPT = E // NS
BLK = 2000
NBLK = EPT // BLK
NGRP = BLK // 16
FB = 80

N0PAD = 50176
RPT0 = N0PAD // NS
ZR = 392
EPAD = 819200
EPC = EPAD // NC
SB = 128
EPT0 = EPC // NS
NSB = EPT0 // SB

_mesh = plsc.VectorSubcoreMesh(core_axis_name="c", subcore_axis_name="s")


def _sc_big_body(g_hbm, row_hbm, col_hbm, zeros_hbm, out_hbm,
                 acc, colb, rowb, pend_d, pend_r,
                 snap_d0, snap_r0, snap_d1, snap_r1, rows0, rows1, gsem):
    c = lax.axis_index("c")
    s = lax.axis_index("s")
    estart = s * EPT

    def _drain(rows_prev, snap_prev):
        pltpu.make_async_copy(g_hbm.at[pl.ds(0, FB)], rows_prev, gsem).wait()
        pltpu.sync_copy(rows_prev, acc.at[snap_prev.at[pl.ds(0, FB)]],
                        add=True)

    def chunk_body(qi, _):
        q = c + NC * qi

        @pl.when(q < NCHUNK)
        def _():
            base = q * CHUNK
            for z in range(CHUNK // NS // 16):
                pltpu.sync_copy(zeros_hbm,
                                acc.at[pl.ds(s * (CHUNK // NS) + z * 16, 16)])
            plsc.subcore_barrier()

            def _flush(snapd, snapr, rows_new, rows_prev, snap_prev, inf):
                @pl.when(jnp.any(inf == 1))
                def _():
                    _drain(rows_prev, snap_prev)
                for k in range(FB // 16):
                    snapd[pl.ds(k * 16, 16)] = pend_d[pl.ds(k * 16, 16)]
                    snapr[pl.ds(k * 16, 16)] = pend_r[pl.ds(k * 16, 16)]
                pltpu.async_copy(g_hbm.at[snapr.at[pl.ds(0, FB)]], rows_new,
                                 gsem)
                pend_d[pl.ds(0, 16)] = pend_d[pl.ds(FB, 16)]
                pend_r[pl.ds(0, 16)] = pend_r[pl.ds(FB, 16)]

            def blk_body(b, st):
                off = estart + b * BLK
                pltpu.sync_copy(row_hbm.at[pl.ds(off, BLK)], rowb)
                pltpu.sync_copy(col_hbm.at[pl.ds(off, BLK)], colb)

                def grp_body(j, st):
                    cnt, par, inf = st
                    c16 = colb[pl.ds(j * 16, 16)]
                    r16 = rowb[pl.ds(j * 16, 16)]
                    lm = c16 - base
                    msk = (lm >= 0) & (lm < CHUNK)
                    mi = msk.astype(jnp.int32)
                    pos = cnt + plsc.cumsum(mi) - 1
                    plsc.store_scatter(pend_d, [pos], lm, mask=msk)
                    plsc.store_scatter(pend_r, [pos], r16, mask=msk)
                    cnt = cnt + plsc.all_reduce_population_count(msk)
                    full = cnt >= FB

                    @pl.when(jnp.any(full & (par == 0)))
                    def _():
                        _flush(snap_d0, snap_r0, rows0, rows1, snap_d1, inf)

                    @pl.when(jnp.any(full & (par == 1)))
                    def _():
                        _flush(snap_d1, snap_r1, rows1, rows0, snap_d0, inf)

                    flush = jnp.any(full)
                    cnt = jnp.where(flush, cnt - FB, cnt)
                    par = jnp.where(flush, 1 - par, par)
                    inf = jnp.where(flush, jnp.ones_like(inf), inf)
                    return (cnt, par, inf)

                return lax.fori_loop(0, NGRP, grp_body, st)

            z16 = jnp.zeros((16,), jnp.int32)
            cnt, par, inf = lax.fori_loop(0, NBLK, blk_body, (z16, z16, z16))

            @pl.when(jnp.any((inf == 1) & (par == 1)))
            def _():
                _drain(rows0, snap_d0)

            @pl.when(jnp.any((inf == 1) & (par == 0)))
            def _():
                _drain(rows1, snap_d1)

            iota16 = lax.iota(jnp.int32, 16)

            def tail_body(g, _):
                lanes = g * 16 + iota16
                valid = lanes < cnt
                idxr = jnp.where(valid, pend_r[pl.ds(g * 16, 16)], -1)
                idxd = jnp.where(valid, pend_d[pl.ds(g * 16, 16)], -1)
                pend_r[pl.ds(240, 16)] = idxr
                pend_d[pl.ds(240, 16)] = idxd
                pltpu.async_copy(
                    g_hbm.at[plsc.Indices(pend_r.at[pl.ds(240, 16)],
                                          ignored_value=-1)],
                    rows0.at[pl.ds(0, 16)], gsem).wait()
                pltpu.sync_copy(
                    rows0.at[pl.ds(0, 16)],
                    acc.at[plsc.Indices(pend_d.at[pl.ds(240, 16)],
                                        ignored_value=-1)],
                    add=True)
                return 0

            lax.fori_loop(0, FB // 16, tail_body, 0)
            plsc.subcore_barrier()
            pltpu.sync_copy(acc.at[pl.ds(s * (CHUNK // NS), CHUNK // NS)],
                            out_hbm.at[pl.ds(base + s * (CHUNK // NS),
                                             CHUNK // NS)])
            plsc.subcore_barrier()

        return 0

    lax.fori_loop(0, CPC, chunk_body, 0)


def _sc_seg_sum_big(g, row, col, zeros16):
    return pl.kernel(
        _sc_big_body,
        out_type=jax.ShapeDtypeStruct((NPAD_BIG, D), jnp.float32),
        mesh=_mesh,
        compiler_params=pltpu.CompilerParams(needs_layout_passes=False,
                                             use_tc_tiling_on_sc=False),
        scratch_types=[
            pltpu.VMEM_SHARED((CHUNK, D), jnp.float32),
            pltpu.VMEM((BLK,), jnp.int32),
            pltpu.VMEM((BLK,), jnp.int32),
            pltpu.VMEM((256,), jnp.int32),
            pltpu.VMEM((256,), jnp.int32),
            pltpu.VMEM((FB,), jnp.int32),
            pltpu.VMEM((FB,), jnp.int32),
            pltpu.VMEM((FB,), jnp.int32),
            pltpu.VMEM((FB,), jnp.int32),
            pltpu.VMEM((FB, D), jnp.float32),
            pltpu.VMEM((FB, D), jnp.float32),
            pltpu.SemaphoreType.DMA,
        ],
    )(g, row, col, zeros16)


def _sc_small_body(g_hbm, row2_hbm, col2_hbm, zeros_hbm, out_hbm,
                   acc, rowb, colb, rows_v, zbuf, gsem):
    c = lax.axis_index("c")
    s = lax.axis_index("s")
    pltpu.sync_copy(zeros_hbm, zbuf)
    for z in range(RPT0 // ZR):
        pltpu.sync_copy(zbuf, acc.at[pl.ds(s * RPT0 + z * ZR, ZR)])
    plsc.subcore_barrier()
    roff = c * (EPC // SB) + s * NSB
    pltpu.sync_copy(row2_hbm.at[pl.ds(roff, NSB)], rowb)
    pltpu.sync_copy(col2_hbm.at[pl.ds(roff, NSB)], colb)

    def sb_body(b, _):
        pltpu.async_copy(g_hbm.at[rowb.at[b]], rows_v, gsem).wait()
        pltpu.sync_copy(rows_v, acc.at[colb.at[b]], add=True)
        return 0

    lax.fori_loop(0, NSB, sb_body, 0)
    plsc.subcore_barrier()
    pltpu.sync_copy(acc.at[pl.ds(s * RPT0, RPT0)],
                    out_hbm.at[c].at[pl.ds(s * RPT0, RPT0)])


def _sc_seg_sum_small(g, row2, col2, zeros_sm):
    return pl.kernel(
        _sc_small_body,
        out_type=jax.ShapeDtypeStruct((NC, N0PAD, D0), jnp.float32),
        mesh=_mesh,
        compiler_params=pltpu.CompilerParams(use_tc_tiling_on_sc=False),
        scratch_types=[
            pltpu.VMEM_SHARED((N0PAD, D0), jnp.float32),
            pltpu.VMEM((NSB, SB), jnp.int32),
            pltpu.VMEM((NSB, SB), jnp.int32),
            pltpu.VMEM((SB, D0), jnp.float32),
            pltpu.VMEM((ZR, D0), jnp.float32),
            pltpu.SemaphoreType.DMA,
        ],
    )(g, row2, col2, zeros_sm)


RB = 1000
RB0 = 1568


def _tc_dis_body(deg2_ref, q_ref, dis_ref, uq_ref):
    dsum = deg2_ref[0] + deg2_ref[1]
    dis = jnp.where(dsum > 0, lax.rsqrt(jnp.maximum(dsum, 1e-12)), 0.0)
    dis_ref[...] = dis
    uq_ref[...] = dis * q_ref[...]


def _tc_dis(deg2, qpad):
    return pl.pallas_call(
        _tc_dis_body,
        grid=(N0PAD // RB0,),
        in_specs=[
            pl.BlockSpec((NC, RB0, D0), lambda i: (0, i, 0)),
            pl.BlockSpec((RB0, D0), lambda i: (i, 0)),
        ],
        out_specs=[
            pl.BlockSpec((RB0, D0), lambda i: (i, 0)),
            pl.BlockSpec((RB0, D0), lambda i: (i, 0)),
        ],
        out_shape=[
            jax.ShapeDtypeStruct((N0PAD, D0), jnp.float32),
            jax.ShapeDtypeStruct((N0PAD, D0), jnp.float32),
        ],
    )(deg2, qpad)


def _tc_w_small_body(v2_ref, dis_ref, w_ref):
    d = dis_ref[...]
    w_ref[...] = d * d * (v2_ref[0] + v2_ref[1])


def _tc_w_small(v2, dis16):
    return pl.pallas_call(
        _tc_w_small_body,
        grid=(N0PAD // RB0,),
        in_specs=[
            pl.BlockSpec((NC, RB0, D0), lambda i: (0, i, 0)),
            pl.BlockSpec((RB0, D0), lambda i: (i, 0)),
        ],
        out_specs=pl.BlockSpec((RB0, D0), lambda i: (i, 0)),
        out_shape=jax.ShapeDtypeStruct((N0PAD, D0), jnp.float32),
    )(v2, dis16)


def _tc_stage0_body(q_ref, dis_ref, v1_ref, v2_ref, v3_ref, g_ref, b0_ref,
                    lng_ref, lnb_ref, x1_ref, u1_ref):
    dis8 = dis_ref[:, :8]
    coef = jnp.concatenate([
        q_ref[:, :8],
        dis8 * (v1_ref[0, :, :8] + v1_ref[1, :, :8]),
        dis8 * (v2_ref[0, :, :8] + v2_ref[1, :, :8]),
        dis8 * (v3_ref[0, :, :8] + v3_ref[1, :, :8]),
    ], axis=1)
    x = jnp.dot(coef, g_ref[...], preferred_element_type=jnp.float32,
                precision=lax.Precision.HIGHEST)
    x = x + b0_ref[...]
    x3 = x.reshape(RB, M, HID)
    mu = jnp.mean(x3, axis=2, keepdims=True)
    xc = x3 - mu
    var = jnp.mean(xc * xc, axis=2, keepdims=True)
    xn = xc * lax.rsqrt(var + 1e-5)
    xn = xn.reshape(RB, D) * lng_ref[...] + lnb_ref[...]
    x1 = jnp.maximum(xn, 0.0)
    x1_ref[...] = x1
    u1_ref[...] = dis_ref[:, :1] * x1


def _tc_stage0(qpad, dis16, vq1, vq2, vq3, G, b0t, lngt, lnbt):
    return pl.pallas_call(
        _tc_stage0_body,
        grid=(N // RB,),
        in_specs=[
            pl.BlockSpec((RB, D0), lambda i: (i, 0)),
            pl.BlockSpec((RB, D0), lambda i: (i, 0)),
            pl.BlockSpec((NC, RB, D0), lambda i: (0, i, 0)),
            pl.BlockSpec((NC, RB, D0), lambda i: (0, i, 0)),
            pl.BlockSpec((NC, RB, D0), lambda i: (0, i, 0)),
            pl.BlockSpec((32, D), lambda i: (0, 0)),
            pl.BlockSpec((1, D), lambda i: (0, 0)),
            pl.BlockSpec((1, D), lambda i: (0, 0)),
            pl.BlockSpec((1, D), lambda i: (0, 0)),
        ],
        out_specs=[
            pl.BlockSpec((RB, D), lambda i: (i, 0)),
            pl.BlockSpec((RB, D), lambda i: (i, 0)),
        ],
        out_shape=[
            jax.ShapeDtypeStruct((N, D), jnp.float32),
            jax.ShapeDtypeStruct((N, D), jnp.float32),
        ],
    )(qpad, dis16, vq1, vq2, vq3, G, b0t, lngt, lnbt)


def _tc_w_big_body(v_ref, dis_ref, w_ref):
    d = dis_ref[:, :1]
    w_ref[...] = d * d * v_ref[...]


def _tc_w_big(v, dis16):
    return pl.pallas_call(
        _tc_w_big_body,
        grid=(N // RB,),
        in_specs=[
            pl.BlockSpec((RB, D), lambda i: (i, 0)),
            pl.BlockSpec((RB, D0), lambda i: (i, 0)),
        ],
        out_specs=pl.BlockSpec((RB, D), lambda i: (i, 0)),
        out_shape=jax.ShapeDtypeStruct((N, D), jnp.float32),
    )(v, dis16)


def _tc_layer1_body(x_ref, v1_ref, v2_ref, v3_ref, dis_ref, bd_ref, b_ref,
                    x2_ref, u2_ref):
    x = x_ref[...]
    dis = dis_ref[:, :1]
    acc = jnp.dot(x, bd_ref[0], preferred_element_type=jnp.float32,
                precision=lax.Precision.HIGHEST)
    s = jnp.dot(v1_ref[...], bd_ref[1], preferred_element_type=jnp.float32,
                precision=lax.Precision.HIGHEST)
    s = s + jnp.dot(v2_ref[...], bd_ref[2], preferred_element_type=jnp.float32,
                precision=lax.Precision.HIGHEST)
    s = s + jnp.dot(v3_ref[...], bd_ref[3], preferred_element_type=jnp.float32,
                precision=lax.Precision.HIGHEST)
    acc = acc + dis * s + b_ref[...]
    x2 = jnp.maximum(acc, 0.0) + x
    x2_ref[...] = x2
    u2_ref[...] = dis * x2


def _tc_layer1(x1, v1, v2, v3, dis16, bd, bt):
    return pl.pallas_call(
        _tc_layer1_body,
        grid=(N // RB,),
        in_specs=[
            pl.BlockSpec((RB, D), lambda i: (i, 0)),
            pl.BlockSpec((RB, D), lambda i: (i, 0)),
            pl.BlockSpec((RB, D), lambda i: (i, 0)),
            pl.BlockSpec((RB, D), lambda i: (i, 0)),
            pl.BlockSpec((RB, D0), lambda i: (i, 0)),
            pl.BlockSpec((4, D, D), lambda i: (0, 0, 0)),
            pl.BlockSpec((1, D), lambda i: (0, 0)),
        ],
        out_specs=[
            pl.BlockSpec((RB, D), lambda i: (i, 0)),
            pl.BlockSpec((RB, D), lambda i: (i, 0)),
        ],
        out_shape=[
            jax.ShapeDtypeStruct((N, D), jnp.float32),
            jax.ShapeDtypeStruct((N, D), jnp.float32),
        ],
    )(x1, v1, v2, v3, dis16, bd, bt)


def _tc_layer2_body(x_ref, v1_ref, v2_ref, v3_ref, dis_ref, bd_ref, b_ref,
                    pt_ref, pb_ref, pooled_ref):
    x = x_ref[...]
    dis = dis_ref[:, :1]
    acc = jnp.dot(x, bd_ref[0], preferred_element_type=jnp.float32,
                precision=lax.Precision.HIGHEST)
    s = jnp.dot(v1_ref[...], bd_ref[1], preferred_element_type=jnp.float32,
                precision=lax.Precision.HIGHEST)
    s = s + jnp.dot(v2_ref[...], bd_ref[2], preferred_element_type=jnp.float32,
                precision=lax.Precision.HIGHEST)
    s = s + jnp.dot(v3_ref[...], bd_ref[3], preferred_element_type=jnp.float32,
                precision=lax.Precision.HIGHEST)
    h3 = acc + dis * s + b_ref[...]
    pooled_ref[...] = (
        jnp.dot(h3, pt_ref[...], preferred_element_type=jnp.float32,
                precision=lax.Precision.HIGHEST)
        + pb_ref[...])


def _tc_layer2(x2, v1, v2, v3, dis16, bd, bt, pt, pjb):
    return pl.pallas_call(
        _tc_layer2_body,
        grid=(N // RB,),
        in_specs=[
            pl.BlockSpec((RB, D), lambda i: (i, 0)),
            pl.BlockSpec((RB, D), lambda i: (i, 0)),
            pl.BlockSpec((RB, D), lambda i: (i, 0)),
            pl.BlockSpec((RB, D), lambda i: (i, 0)),
            pl.BlockSpec((RB, D0), lambda i: (i, 0)),
            pl.BlockSpec((4, D, D), lambda i: (0, 0, 0)),
            pl.BlockSpec((1, D), lambda i: (0, 0)),
            pl.BlockSpec((D, DM), lambda i: (0, 0)),
            pl.BlockSpec((1, DM), lambda i: (0, 0)),
        ],
        out_specs=pl.BlockSpec((RB, DM), lambda i: (i, 0)),
        out_shape=jax.ShapeDtypeStruct((N, DM), jnp.float32),
    )(x2, v1, v2, v3, dis16, bd, bt, pt, pjb)


def _tc_bn_sum_body(p_ref, s_ref):
    @pl.when(pl.program_id(0) == 0)
    def _():
        s_ref[...] = jnp.zeros_like(s_ref)

    s_ref[...] += jnp.sum(p_ref[...], axis=0, keepdims=True)


def _tc_bn_sum(pooled):
    return pl.pallas_call(
        _tc_bn_sum_body,
        grid=(N // RB,),
        in_specs=[pl.BlockSpec((RB, DM), lambda i: (i, 0))],
        out_specs=pl.BlockSpec((1, DM), lambda i: (0, 0)),
        out_shape=jax.ShapeDtypeStruct((1, DM), jnp.float32),
    )(pooled)


def _tc_bn_var_body(p_ref, mu_ref, s_ref):
    @pl.when(pl.program_id(0) == 0)
    def _():
        s_ref[...] = jnp.zeros_like(s_ref)

    d = p_ref[...] - mu_ref[...]
    s_ref[...] += jnp.sum(d * d, axis=0, keepdims=True)


def _tc_bn_var(pooled, mu):
    return pl.pallas_call(
        _tc_bn_var_body,
        grid=(N // RB,),
        in_specs=[
            pl.BlockSpec((RB, DM), lambda i: (i, 0)),
            pl.BlockSpec((1, DM), lambda i: (0, 0)),
        ],
        out_specs=pl.BlockSpec((1, DM), lambda i: (0, 0)),
        out_shape=jax.ShapeDtypeStruct((1, DM), jnp.float32),
    )(pooled, mu)


def _tc_bn_norm_body(p_ref, mu_ref, rstd_ref, g_ref, b_ref, out_ref):
    out_ref[...] = ((p_ref[...] - mu_ref[...]) * rstd_ref[...] * g_ref[...]
                    + b_ref[...])


def _tc_bn_norm(pooled, mu, rstd, g, b):
    return pl.pallas_call(
        _tc_bn_norm_body,
        grid=(N // RB,),
        in_specs=[
            pl.BlockSpec((RB, DM), lambda i: (i, 0)),
            pl.BlockSpec((1, DM), lambda i: (0, 0)),
            pl.BlockSpec((1, DM), lambda i: (0, 0)),
            pl.BlockSpec((1, DM), lambda i: (0, 0)),
            pl.BlockSpec((1, DM), lambda i: (0, 0)),
        ],
        out_specs=pl.BlockSpec((RB, DM), lambda i: (i, 0)),
        out_shape=jax.ShapeDtypeStruct((N, DM), jnp.float32),
    )(pooled, mu, rstd, g, b)


def kernel(x, edge_index, Q, conv0_W, conv0_b, conv1_W, conv1_b, conv2_W,
           conv2_b, ln_g, ln_b, proj_W, proj_b, bn_g, bn_b):
    row = edge_index[0]
    col = edge_index[1]

    pad = EPAD - E
    row2 = jnp.concatenate(
        [row, jnp.zeros((pad,), jnp.int32)]).reshape(EPAD // SB, SB)
    col2 = jnp.concatenate(
        [col, jnp.full((pad,), N, jnp.int32)]).reshape(EPAD // SB, SB)
    qpad = jnp.zeros((N0PAD, D0), jnp.float32).at[:N, :M].set(Q)
    ones_tbl = jnp.ones((N0PAD, D0), jnp.float32)
    zeros_sm = jnp.zeros((ZR, D0), jnp.float32)
    zeros16 = jnp.zeros((16, D), jnp.float32)

    eye8 = jnp.eye(M, dtype=jnp.float32)
    G = (eye8[None, :, :, None] * conv0_W[:, 0][:, None, None, :]).reshape(
        4 * M, D)
    b0t = jnp.tile(conv0_b, M)[None]
    lngt = jnp.tile(ln_g, M)[None]
    lnbt = jnp.tile(ln_b, M)[None]
    bd1 = jnp.stack([jnp.kron(eye8, conv1_W[k]) for k in range(4)])
    b1t = jnp.tile(conv1_b, M)[None]
    bd2 = jnp.stack([jnp.kron(eye8, conv2_W[k]) for k in range(4)])
    b2t = jnp.tile(conv2_b, M)[None]
    pt = jnp.tile(proj_W, (M, 1)) / M
    pjb = proj_b[None]

    deg2 = _sc_seg_sum_small(ones_tbl, row2, col2, zeros_sm)
    dis16, uq = _tc_dis(deg2, qpad)

    vq1 = _sc_seg_sum_small(uq, row2, col2, zeros_sm)
    wq1 = _tc_w_small(vq1, dis16)
    vq2 = _sc_seg_sum_small(wq1, row2, col2, zeros_sm)
    wq2 = _tc_w_small(vq2, dis16)
    vq3 = _sc_seg_sum_small(wq2, row2, col2, zeros_sm)
    x1, u1 = _tc_stage0(qpad, dis16, vq1, vq2, vq3, G, b0t, lngt, lnbt)

    v11 = _sc_seg_sum_big(u1, row, col, zeros16)
    w11 = _tc_w_big(v11, dis16)
    v12 = _sc_seg_sum_big(w11, row, col, zeros16)
    w12 = _tc_w_big(v12, dis16)
    v13 = _sc_seg_sum_big(w12, row, col, zeros16)
    x2, u2 = _tc_layer1(x1, v11, v12, v13, dis16, bd1, b1t)

    v21 = _sc_seg_sum_big(u2, row, col, zeros16)
    w21 = _tc_w_big(v21, dis16)
    v22 = _sc_seg_sum_big(w21, row, col, zeros16)
    w22 = _tc_w_big(v22, dis16)
    v23 = _sc_seg_sum_big(w22, row, col, zeros16)
    pooled = _tc_layer2(x2, v21, v22, v23, dis16, bd2, b2t, pt, pjb)

    mu = _tc_bn_sum(pooled) / N
    var = _tc_bn_var(pooled, mu) / N
    rstd = lax.rsqrt(var + 1e-5)
    return _tc_bn_norm(pooled, mu, rstd, bn_g[None], bn_b[None])


# speedup vs baseline: 1.9243x; 1.9243x over previous
import jax
import jax.numpy as jnp
from jax import lax
from jax.experimental import pallas as pl
from jax.experimental.pallas import tpu as pltpu
from jax.experimental.pallas import tpu_sc as plsc

N = 50000
E = 800000
HID = 64
DM = 128
M = 8
D = 512
D0 = 16

NC = 2
NS = 16

CHUNK = 1280
NCHUNK = -(-N // CHUNK)
NPAD_BIG = NCHUNK * CHUNK
CPC = -(-NCHUNK // NC)
EPT = E // NS
BLK = 2000
NBLK = EPT // BLK
NGRP = BLK // 16
FB = 80

N0PAD = 50176
RPT0 = N0PAD // NS
ZR = 392
EPAD = 819200
EPC = EPAD // NC
SB = 128
EPT0 = EPC // NS
NSB = EPT0 // SB

_mesh = plsc.VectorSubcoreMesh(core_axis_name="c", subcore_axis_name="s")


def _sc_big_body(g_hbm, row_hbm, col_hbm, zeros_hbm, out_hbm,
                 acc, colb, rowb, pend_d, pend_r,
                 snap_d0, snap_r0, snap_d1, snap_r1, rows0, rows1, gsem):
    c = lax.axis_index("c")
    s = lax.axis_index("s")
    estart = s * EPT

    def _drain(rows_prev, snap_prev):
        pltpu.make_async_copy(g_hbm.at[pl.ds(0, FB)], rows_prev, gsem).wait()
        pltpu.sync_copy(rows_prev, acc.at[snap_prev.at[pl.ds(0, FB)]],
                        add=True)

    def chunk_body(qi, _):
        q = c + NC * qi

        @pl.when(q < NCHUNK)
        def _():
            base = q * CHUNK
            for z in range(CHUNK // NS // 16):
                pltpu.sync_copy(zeros_hbm,
                                acc.at[pl.ds(s * (CHUNK // NS) + z * 16, 16)])
            plsc.subcore_barrier()

            def _flush(snapd, snapr, rows_new, rows_prev, snap_prev, inf):
                @pl.when(jnp.any(inf == 1))
                def _():
                    _drain(rows_prev, snap_prev)
                for k in range(FB // 16):
                    snapd[pl.ds(k * 16, 16)] = pend_d[pl.ds(k * 16, 16)]
                    snapr[pl.ds(k * 16, 16)] = pend_r[pl.ds(k * 16, 16)]
                pltpu.async_copy(g_hbm.at[snapr.at[pl.ds(0, FB)]], rows_new,
                                 gsem)
                for k in range(8):
                    pend_d[pl.ds(k * 16, 16)] = pend_d[pl.ds(FB + k * 16, 16)]
                    pend_r[pl.ds(k * 16, 16)] = pend_r[pl.ds(FB + k * 16, 16)]

            def _compact(j0, cnt):
                c16 = colb[pl.ds(j0, 16)]
                r16 = rowb[pl.ds(j0, 16)]
                lm = c16 - base
                msk = (lm >= 0) & (lm < CHUNK)
                mi = msk.astype(jnp.int32)
                pos = cnt + plsc.cumsum(mi) - 1
                plsc.store_scatter(pend_d, [pos], lm, mask=msk)
                plsc.store_scatter(pend_r, [pos], r16, mask=msk)
                return cnt + plsc.all_reduce_population_count(msk)

            def _flush_round(st):
                cnt, par, inf = st
                fl = jnp.any(cnt >= FB)

                @pl.when(fl)
                def _():
                    @pl.when(jnp.any(par == 0))
                    def _():
                        _flush(snap_d0, snap_r0, rows0, rows1, snap_d1, inf)

                    @pl.when(jnp.any(par == 1))
                    def _():
                        _flush(snap_d1, snap_r1, rows1, rows0, snap_d0, inf)

                cnt = jnp.where(fl, cnt - FB, cnt)
                par = jnp.where(fl, 1 - par, par)
                inf = jnp.where(fl, jnp.ones_like(inf), inf)
                return (cnt, par, inf)

            def blk_body(b, st):
                off = estart + b * BLK
                pltpu.sync_copy(row_hbm.at[pl.ds(off, BLK)], rowb)
                pltpu.sync_copy(col_hbm.at[pl.ds(off, BLK)], colb)

                def sgrp_body(jj, st):
                    cnt, par, inf = st
                    for u in range(8):
                        cnt = _compact(jj * 128 + u * 16, cnt)
                    st = _flush_round((cnt, par, inf))
                    return _flush_round(st)

                st = lax.fori_loop(0, NGRP // 8, sgrp_body, st)
                cnt, par, inf = st
                for u in range(NGRP % 8):
                    cnt = _compact((NGRP // 8) * 128 + u * 16, cnt)
                st = _flush_round((cnt, par, inf))
                return _flush_round(st)

            z16 = jnp.zeros((16,), jnp.int32)
            cnt, par, inf = lax.fori_loop(0, NBLK, blk_body, (z16, z16, z16))

            @pl.when(jnp.any((inf == 1) & (par == 1)))
            def _():
                _drain(rows0, snap_d0)

            @pl.when(jnp.any((inf == 1) & (par == 0)))
            def _():
                _drain(rows1, snap_d1)

            iota16 = lax.iota(jnp.int32, 16)

            def tail_body(g, _):
                lanes = g * 16 + iota16
                valid = lanes < cnt
                idxr = jnp.where(valid, pend_r[pl.ds(g * 16, 16)], -1)
                idxd = jnp.where(valid, pend_d[pl.ds(g * 16, 16)], -1)
                pend_r[pl.ds(240, 16)] = idxr
                pend_d[pl.ds(240, 16)] = idxd
                pltpu.async_copy(
                    g_hbm.at[plsc.Indices(pend_r.at[pl.ds(240, 16)],
                                          ignored_value=-1)],
                    rows0.at[pl.ds(0, 16)], gsem).wait()
                pltpu.sync_copy(
                    rows0.at[pl.ds(0, 16)],
                    acc.at[plsc.Indices(pend_d.at[pl.ds(240, 16)],
                                        ignored_value=-1)],
                    add=True)
                return 0

            lax.fori_loop(0, FB // 16, tail_body, 0)
            plsc.subcore_barrier()
            pltpu.sync_copy(acc.at[pl.ds(s * (CHUNK // NS), CHUNK // NS)],
                            out_hbm.at[pl.ds(base + s * (CHUNK // NS),
                                             CHUNK // NS)])
            plsc.subcore_barrier()

        return 0

    lax.fori_loop(0, CPC, chunk_body, 0)


def _sc_seg_sum_big(g, row, col, zeros16):
    return pl.kernel(
        _sc_big_body,
        out_type=jax.ShapeDtypeStruct((NPAD_BIG, D), jnp.float32),
        mesh=_mesh,
        compiler_params=pltpu.CompilerParams(needs_layout_passes=False,
                                             use_tc_tiling_on_sc=False),
        scratch_types=[
            pltpu.VMEM_SHARED((CHUNK, D), jnp.float32),
            pltpu.VMEM((BLK,), jnp.int32),
            pltpu.VMEM((BLK,), jnp.int32),
            pltpu.VMEM((256,), jnp.int32),
            pltpu.VMEM((256,), jnp.int32),
            pltpu.VMEM((FB,), jnp.int32),
            pltpu.VMEM((FB,), jnp.int32),
            pltpu.VMEM((FB,), jnp.int32),
            pltpu.VMEM((FB,), jnp.int32),
            pltpu.VMEM((FB, D), jnp.float32),
            pltpu.VMEM((FB, D), jnp.float32),
            pltpu.SemaphoreType.DMA,
        ],
    )(g, row, col, zeros16)


def _sc_small_body(g_hbm, row2_hbm, col2_hbm, zeros_hbm, out_hbm,
                   acc, rowb, colb, rows_v, zbuf, gsem):
    c = lax.axis_index("c")
    s = lax.axis_index("s")
    pltpu.sync_copy(zeros_hbm, zbuf)
    for z in range(RPT0 // ZR):
        pltpu.sync_copy(zbuf, acc.at[pl.ds(s * RPT0 + z * ZR, ZR)])
    plsc.subcore_barrier()
    roff = c * (EPC // SB) + s * NSB
    pltpu.sync_copy(row2_hbm.at[pl.ds(roff, NSB)], rowb)
    pltpu.sync_copy(col2_hbm.at[pl.ds(roff, NSB)], colb)

    def sb_body(b, _):
        pltpu.async_copy(g_hbm.at[rowb.at[b]], rows_v, gsem).wait()
        pltpu.sync_copy(rows_v, acc.at[colb.at[b]], add=True)
        return 0

    lax.fori_loop(0, NSB, sb_body, 0)
    plsc.subcore_barrier()
    pltpu.sync_copy(acc.at[pl.ds(s * RPT0, RPT0)],
                    out_hbm.at[c].at[pl.ds(s * RPT0, RPT0)])


def _sc_seg_sum_small(g, row2, col2, zeros_sm):
    return pl.kernel(
        _sc_small_body,
        out_type=jax.ShapeDtypeStruct((NC, N0PAD, D0), jnp.float32),
        mesh=_mesh,
        compiler_params=pltpu.CompilerParams(use_tc_tiling_on_sc=False),
        scratch_types=[
            pltpu.VMEM_SHARED((N0PAD, D0), jnp.float32),
            pltpu.VMEM((NSB, SB), jnp.int32),
            pltpu.VMEM((NSB, SB), jnp.int32),
            pltpu.VMEM((SB, D0), jnp.float32),
            pltpu.VMEM((ZR, D0), jnp.float32),
            pltpu.SemaphoreType.DMA,
        ],
    )(g, row2, col2, zeros_sm)


RB = 1000
RB0 = 1568


def _tc_dis_body(deg2_ref, q_ref, dis_ref, uq_ref):
    dsum = deg2_ref[0] + deg2_ref[1]
    dis = jnp.where(dsum > 0, lax.rsqrt(jnp.maximum(dsum, 1e-12)), 0.0)
    dis_ref[...] = dis
    uq_ref[...] = dis * q_ref[...]


def _tc_dis(deg2, qpad):
    return pl.pallas_call(
        _tc_dis_body,
        grid=(N0PAD // RB0,),
        in_specs=[
            pl.BlockSpec((NC, RB0, D0), lambda i: (0, i, 0)),
            pl.BlockSpec((RB0, D0), lambda i: (i, 0)),
        ],
        out_specs=[
            pl.BlockSpec((RB0, D0), lambda i: (i, 0)),
            pl.BlockSpec((RB0, D0), lambda i: (i, 0)),
        ],
        out_shape=[
            jax.ShapeDtypeStruct((N0PAD, D0), jnp.float32),
            jax.ShapeDtypeStruct((N0PAD, D0), jnp.float32),
        ],
    )(deg2, qpad)


def _tc_w_small_body(v2_ref, dis_ref, w_ref):
    d = dis_ref[...]
    w_ref[...] = d * d * (v2_ref[0] + v2_ref[1])


def _tc_w_small(v2, dis16):
    return pl.pallas_call(
        _tc_w_small_body,
        grid=(N0PAD // RB0,),
        in_specs=[
            pl.BlockSpec((NC, RB0, D0), lambda i: (0, i, 0)),
            pl.BlockSpec((RB0, D0), lambda i: (i, 0)),
        ],
        out_specs=pl.BlockSpec((RB0, D0), lambda i: (i, 0)),
        out_shape=jax.ShapeDtypeStruct((N0PAD, D0), jnp.float32),
    )(v2, dis16)


def _tc_stage0_body(q_ref, dis_ref, v1_ref, v2_ref, v3_ref, g_ref, b0_ref,
                    lng_ref, lnb_ref, x1_ref, u1_ref):
    dis8 = dis_ref[:, :8]
    coef = jnp.concatenate([
        q_ref[:, :8],
        dis8 * (v1_ref[0, :, :8] + v1_ref[1, :, :8]),
        dis8 * (v2_ref[0, :, :8] + v2_ref[1, :, :8]),
        dis8 * (v3_ref[0, :, :8] + v3_ref[1, :, :8]),
    ], axis=1)
    x = jnp.dot(coef, g_ref[...], preferred_element_type=jnp.float32,
                precision=lax.Precision.HIGHEST)
    x = x + b0_ref[...]
    x3 = x.reshape(RB, M, HID)
    mu = jnp.mean(x3, axis=2, keepdims=True)
    xc = x3 - mu
    var = jnp.mean(xc * xc, axis=2, keepdims=True)
    xn = xc * lax.rsqrt(var + 1e-5)
    xn = xn.reshape(RB, D) * lng_ref[...] + lnb_ref[...]
    x1 = jnp.maximum(xn, 0.0)
    x1_ref[...] = x1
    u1_ref[...] = dis_ref[:, :1] * x1


def _tc_stage0(qpad, dis16, vq1, vq2, vq3, G, b0t, lngt, lnbt):
    return pl.pallas_call(
        _tc_stage0_body,
        grid=(N // RB,),
        in_specs=[
            pl.BlockSpec((RB, D0), lambda i: (i, 0)),
            pl.BlockSpec((RB, D0), lambda i: (i, 0)),
            pl.BlockSpec((NC, RB, D0), lambda i: (0, i, 0)),
            pl.BlockSpec((NC, RB, D0), lambda i: (0, i, 0)),
            pl.BlockSpec((NC, RB, D0), lambda i: (0, i, 0)),
            pl.BlockSpec((32, D), lambda i: (0, 0)),
            pl.BlockSpec((1, D), lambda i: (0, 0)),
            pl.BlockSpec((1, D), lambda i: (0, 0)),
            pl.BlockSpec((1, D), lambda i: (0, 0)),
        ],
        out_specs=[
            pl.BlockSpec((RB, D), lambda i: (i, 0)),
            pl.BlockSpec((RB, D), lambda i: (i, 0)),
        ],
        out_shape=[
            jax.ShapeDtypeStruct((N, D), jnp.float32),
            jax.ShapeDtypeStruct((N, D), jnp.float32),
        ],
    )(qpad, dis16, vq1, vq2, vq3, G, b0t, lngt, lnbt)


def _tc_w_big_body(v_ref, dis_ref, w_ref):
    d = dis_ref[:, :1]
    w_ref[...] = d * d * v_ref[...]


def _tc_w_big(v, dis16):
    return pl.pallas_call(
        _tc_w_big_body,
        grid=(N // RB,),
        in_specs=[
            pl.BlockSpec((RB, D), lambda i: (i, 0)),
            pl.BlockSpec((RB, D0), lambda i: (i, 0)),
        ],
        out_specs=pl.BlockSpec((RB, D), lambda i: (i, 0)),
        out_shape=jax.ShapeDtypeStruct((N, D), jnp.float32),
    )(v, dis16)


def _tc_layer1_body(x_ref, v1_ref, v2_ref, v3_ref, dis_ref, bd_ref, b_ref,
                    x2_ref, u2_ref):
    x = x_ref[...]
    dis = dis_ref[:, :1]
    acc = jnp.dot(x, bd_ref[0], preferred_element_type=jnp.float32,
                precision=lax.Precision.HIGHEST)
    s = jnp.dot(v1_ref[...], bd_ref[1], preferred_element_type=jnp.float32,
                precision=lax.Precision.HIGHEST)
    s = s + jnp.dot(v2_ref[...], bd_ref[2], preferred_element_type=jnp.float32,
                precision=lax.Precision.HIGHEST)
    s = s + jnp.dot(v3_ref[...], bd_ref[3], preferred_element_type=jnp.float32,
                precision=lax.Precision.HIGHEST)
    acc = acc + dis * s + b_ref[...]
    x2 = jnp.maximum(acc, 0.0) + x
    x2_ref[...] = x2
    u2_ref[...] = dis * x2


def _tc_layer1(x1, v1, v2, v3, dis16, bd, bt):
    return pl.pallas_call(
        _tc_layer1_body,
        grid=(N // RB,),
        in_specs=[
            pl.BlockSpec((RB, D), lambda i: (i, 0)),
            pl.BlockSpec((RB, D), lambda i: (i, 0)),
            pl.BlockSpec((RB, D), lambda i: (i, 0)),
            pl.BlockSpec((RB, D), lambda i: (i, 0)),
            pl.BlockSpec((RB, D0), lambda i: (i, 0)),
            pl.BlockSpec((4, D, D), lambda i: (0, 0, 0)),
            pl.BlockSpec((1, D), lambda i: (0, 0)),
        ],
        out_specs=[
            pl.BlockSpec((RB, D), lambda i: (i, 0)),
            pl.BlockSpec((RB, D), lambda i: (i, 0)),
        ],
        out_shape=[
            jax.ShapeDtypeStruct((N, D), jnp.float32),
            jax.ShapeDtypeStruct((N, D), jnp.float32),
        ],
    )(x1, v1, v2, v3, dis16, bd, bt)


def _tc_layer2_body(x_ref, v1_ref, v2_ref, v3_ref, dis_ref, bd_ref, b_ref,
                    pt_ref, pb_ref, pooled_ref):
    x = x_ref[...]
    dis = dis_ref[:, :1]
    acc = jnp.dot(x, bd_ref[0], preferred_element_type=jnp.float32,
                precision=lax.Precision.HIGHEST)
    s = jnp.dot(v1_ref[...], bd_ref[1], preferred_element_type=jnp.float32,
                precision=lax.Precision.HIGHEST)
    s = s + jnp.dot(v2_ref[...], bd_ref[2], preferred_element_type=jnp.float32,
                precision=lax.Precision.HIGHEST)
    s = s + jnp.dot(v3_ref[...], bd_ref[3], preferred_element_type=jnp.float32,
                precision=lax.Precision.HIGHEST)
    h3 = acc + dis * s + b_ref[...]
    pooled_ref[...] = (
        jnp.dot(h3, pt_ref[...], preferred_element_type=jnp.float32,
                precision=lax.Precision.HIGHEST)
        + pb_ref[...])


def _tc_layer2(x2, v1, v2, v3, dis16, bd, bt, pt, pjb):
    return pl.pallas_call(
        _tc_layer2_body,
        grid=(N // RB,),
        in_specs=[
            pl.BlockSpec((RB, D), lambda i: (i, 0)),
            pl.BlockSpec((RB, D), lambda i: (i, 0)),
            pl.BlockSpec((RB, D), lambda i: (i, 0)),
            pl.BlockSpec((RB, D), lambda i: (i, 0)),
            pl.BlockSpec((RB, D0), lambda i: (i, 0)),
            pl.BlockSpec((4, D, D), lambda i: (0, 0, 0)),
            pl.BlockSpec((1, D), lambda i: (0, 0)),
            pl.BlockSpec((D, DM), lambda i: (0, 0)),
            pl.BlockSpec((1, DM), lambda i: (0, 0)),
        ],
        out_specs=pl.BlockSpec((RB, DM), lambda i: (i, 0)),
        out_shape=jax.ShapeDtypeStruct((N, DM), jnp.float32),
    )(x2, v1, v2, v3, dis16, bd, bt, pt, pjb)


def _tc_bn_sum_body(p_ref, s_ref):
    @pl.when(pl.program_id(0) == 0)
    def _():
        s_ref[...] = jnp.zeros_like(s_ref)

    s_ref[...] += jnp.sum(p_ref[...], axis=0, keepdims=True)


def _tc_bn_sum(pooled):
    return pl.pallas_call(
        _tc_bn_sum_body,
        grid=(N // RB,),
        in_specs=[pl.BlockSpec((RB, DM), lambda i: (i, 0))],
        out_specs=pl.BlockSpec((1, DM), lambda i: (0, 0)),
        out_shape=jax.ShapeDtypeStruct((1, DM), jnp.float32),
    )(pooled)


def _tc_bn_var_body(p_ref, mu_ref, s_ref):
    @pl.when(pl.program_id(0) == 0)
    def _():
        s_ref[...] = jnp.zeros_like(s_ref)

    d = p_ref[...] - mu_ref[...]
    s_ref[...] += jnp.sum(d * d, axis=0, keepdims=True)


def _tc_bn_var(pooled, mu):
    return pl.pallas_call(
        _tc_bn_var_body,
        grid=(N // RB,),
        in_specs=[
            pl.BlockSpec((RB, DM), lambda i: (i, 0)),
            pl.BlockSpec((1, DM), lambda i: (0, 0)),
        ],
        out_specs=pl.BlockSpec((1, DM), lambda i: (0, 0)),
        out_shape=jax.ShapeDtypeStruct((1, DM), jnp.float32),
    )(pooled, mu)


def _tc_bn_norm_body(p_ref, mu_ref, rstd_ref, g_ref, b_ref, out_ref):
    out_ref[...] = ((p_ref[...] - mu_ref[...]) * rstd_ref[...] * g_ref[...]
                    + b_ref[...])


def _tc_bn_norm(pooled, mu, rstd, g, b):
    return pl.pallas_call(
        _tc_bn_norm_body,
        grid=(N // RB,),
        in_specs=[
            pl.BlockSpec((RB, DM), lambda i: (i, 0)),
            pl.BlockSpec((1, DM), lambda i: (0, 0)),
            pl.BlockSpec((1, DM), lambda i: (0, 0)),
            pl.BlockSpec((1, DM), lambda i: (0, 0)),
            pl.BlockSpec((1, DM), lambda i: (0, 0)),
        ],
        out_specs=pl.BlockSpec((RB, DM), lambda i: (i, 0)),
        out_shape=jax.ShapeDtypeStruct((N, DM), jnp.float32),
    )(pooled, mu, rstd, g, b)


def kernel(x, edge_index, Q, conv0_W, conv0_b, conv1_W, conv1_b, conv2_W,
           conv2_b, ln_g, ln_b, proj_W, proj_b, bn_g, bn_b):
    row = edge_index[0]
    col = edge_index[1]

    pad = EPAD - E
    row2 = jnp.concatenate(
        [row, jnp.zeros((pad,), jnp.int32)]).reshape(EPAD // SB, SB)
    col2 = jnp.concatenate(
        [col, jnp.full((pad,), N, jnp.int32)]).reshape(EPAD // SB, SB)
    qpad = jnp.zeros((N0PAD, D0), jnp.float32).at[:N, :M].set(Q)
    ones_tbl = jnp.ones((N0PAD, D0), jnp.float32)
    zeros_sm = jnp.zeros((ZR, D0), jnp.float32)
    zeros16 = jnp.zeros((16, D), jnp.float32)

    eye8 = jnp.eye(M, dtype=jnp.float32)
    G = (eye8[None, :, :, None] * conv0_W[:, 0][:, None, None, :]).reshape(
        4 * M, D)
    b0t = jnp.tile(conv0_b, M)[None]
    lngt = jnp.tile(ln_g, M)[None]
    lnbt = jnp.tile(ln_b, M)[None]
    bd1 = jnp.stack([jnp.kron(eye8, conv1_W[k]) for k in range(4)])
    b1t = jnp.tile(conv1_b, M)[None]
    bd2 = jnp.stack([jnp.kron(eye8, conv2_W[k]) for k in range(4)])
    b2t = jnp.tile(conv2_b, M)[None]
    pt = jnp.tile(proj_W, (M, 1)) / M
    pjb = proj_b[None]

    deg2 = _sc_seg_sum_small(ones_tbl, row2, col2, zeros_sm)
    dis16, uq = _tc_dis(deg2, qpad)

    vq1 = _sc_seg_sum_small(uq, row2, col2, zeros_sm)
    wq1 = _tc_w_small(vq1, dis16)
    vq2 = _sc_seg_sum_small(wq1, row2, col2, zeros_sm)
    wq2 = _tc_w_small(vq2, dis16)
    vq3 = _sc_seg_sum_small(wq2, row2, col2, zeros_sm)
    x1, u1 = _tc_stage0(qpad, dis16, vq1, vq2, vq3, G, b0t, lngt, lnbt)

    v11 = _sc_seg_sum_big(u1, row, col, zeros16)
    w11 = _tc_w_big(v11, dis16)
    v12 = _sc_seg_sum_big(w11, row, col, zeros16)
    w12 = _tc_w_big(v12, dis16)
    v13 = _sc_seg_sum_big(w12, row, col, zeros16)
    x2, u2 = _tc_layer1(x1, v11, v12, v13, dis16, bd1, b1t)

    v21 = _sc_seg_sum_big(u2, row, col, zeros16)
    w21 = _tc_w_big(v21, dis16)
    v22 = _sc_seg_sum_big(w21, row, col, zeros16)
    w22 = _tc_w_big(v22, dis16)
    v23 = _sc_seg_sum_big(w22, row, col, zeros16)
    pooled = _tc_layer2(x2, v21, v22, v23, dis16, bd2, b2t, pt, pjb)

    mu = _tc_bn_sum(pooled) / N
    var = _tc_bn_var(pooled, mu) / N
    rstd = lax.rsqrt(var + 1e-5)
    return _tc_bn_norm(pooled, mu, rstd, bn_g[None], bn_b[None])


# speedup vs baseline: 2.1009x; 1.0918x over previous
import jax
import jax.numpy as jnp
from jax import lax
from jax.experimental import pallas as pl
from jax.experimental.pallas import tpu as pltpu
from jax.experimental.pallas import tpu_sc as plsc

N = 50000
E = 800000
HID = 64
DM = 128
M = 8
D = 512
D0 = 16

NC = 2
NS = 16

CHUNK = 1536
NCHUNK = -(-N // CHUNK)
NPAD_BIG = NCHUNK * CHUNK
CPC = -(-NCHUNK // NC)
EPT = E // NS
BLK = 2000
NBLK = EPT // BLK
NGRP = BLK // 16
FB = 64

N0PAD = 50176
RPT0 = N0PAD // NS
ZR = 392
EPAD = 819200
EPC = EPAD // NC
SB = 128
EPT0 = EPC // NS
NSB = EPT0 // SB

_mesh = plsc.VectorSubcoreMesh(core_axis_name="c", subcore_axis_name="s")


def _sc_big_body(g_hbm, row_hbm, col_hbm, zeros_hbm, out_hbm,
                 acc, colb, rowb, pend_d, pend_r,
                 snap_d0, snap_r0, snap_d1, snap_r1, rows0, rows1, gsem):
    c = lax.axis_index("c")
    s = lax.axis_index("s")
    estart = s * EPT

    def _drain(rows_prev, snap_prev):
        pltpu.make_async_copy(g_hbm.at[pl.ds(0, FB)], rows_prev, gsem).wait()
        pltpu.sync_copy(rows_prev, acc.at[snap_prev.at[pl.ds(0, FB)]],
                        add=True)

    def chunk_body(qi, _):
        q = c + NC * qi

        @pl.when(q < NCHUNK)
        def _():
            base = q * CHUNK
            for z in range(CHUNK // NS // 16):
                pltpu.sync_copy(zeros_hbm,
                                acc.at[pl.ds(s * (CHUNK // NS) + z * 16, 16)])
            plsc.subcore_barrier()

            def _flush(snapd, snapr, rows_new, rows_prev, snap_prev, inf):
                @pl.when(jnp.any(inf == 1))
                def _():
                    _drain(rows_prev, snap_prev)
                for k in range(FB // 16):
                    snapd[pl.ds(k * 16, 16)] = pend_d[pl.ds(k * 16, 16)]
                    snapr[pl.ds(k * 16, 16)] = pend_r[pl.ds(k * 16, 16)]
                pltpu.async_copy(g_hbm.at[snapr.at[pl.ds(0, FB)]], rows_new,
                                 gsem)
                for k in range(8):
                    pend_d[pl.ds(k * 16, 16)] = pend_d[pl.ds(FB + k * 16, 16)]
                    pend_r[pl.ds(k * 16, 16)] = pend_r[pl.ds(FB + k * 16, 16)]

            def _compact(j0, cnt):
                c16 = colb[pl.ds(j0, 16)]
                r16 = rowb[pl.ds(j0, 16)]
                lm = c16 - base
                msk = (lm >= 0) & (lm < CHUNK)
                mi = msk.astype(jnp.int32)
                pos = cnt + plsc.cumsum(mi) - 1
                plsc.store_scatter(pend_d, [pos], lm, mask=msk)
                plsc.store_scatter(pend_r, [pos], r16, mask=msk)
                return cnt + plsc.all_reduce_population_count(msk)

            def _flush_round(st):
                cnt, par, inf = st
                fl = jnp.any(cnt >= FB)

                @pl.when(fl)
                def _():
                    @pl.when(jnp.any(par == 0))
                    def _():
                        _flush(snap_d0, snap_r0, rows0, rows1, snap_d1, inf)

                    @pl.when(jnp.any(par == 1))
                    def _():
                        _flush(snap_d1, snap_r1, rows1, rows0, snap_d0, inf)

                cnt = jnp.where(fl, cnt - FB, cnt)
                par = jnp.where(fl, 1 - par, par)
                inf = jnp.where(fl, jnp.ones_like(inf), inf)
                return (cnt, par, inf)

            def blk_body(b, st):
                off = estart + b * BLK
                pltpu.sync_copy(row_hbm.at[pl.ds(off, BLK)], rowb)
                pltpu.sync_copy(col_hbm.at[pl.ds(off, BLK)], colb)

                def sgrp_body(jj, st):
                    cnt, par, inf = st
                    for u in range(8):
                        cnt = _compact(jj * 128 + u * 16, cnt)
                    st = _flush_round((cnt, par, inf))
                    return _flush_round(st)

                st = lax.fori_loop(0, NGRP // 8, sgrp_body, st)
                cnt, par, inf = st
                for u in range(NGRP % 8):
                    cnt = _compact((NGRP // 8) * 128 + u * 16, cnt)
                st = _flush_round((cnt, par, inf))
                return _flush_round(st)

            z16 = jnp.zeros((16,), jnp.int32)
            cnt, par, inf = lax.fori_loop(0, NBLK, blk_body, (z16, z16, z16))

            @pl.when(jnp.any((inf == 1) & (par == 1)))
            def _():
                _drain(rows0, snap_d0)

            @pl.when(jnp.any((inf == 1) & (par == 0)))
            def _():
                _drain(rows1, snap_d1)

            iota16 = lax.iota(jnp.int32, 16)

            def tail_body(g, _):
                lanes = g * 16 + iota16
                valid = lanes < cnt
                idxr = jnp.where(valid, pend_r[pl.ds(g * 16, 16)], -1)
                idxd = jnp.where(valid, pend_d[pl.ds(g * 16, 16)], -1)
                pend_r[pl.ds(240, 16)] = idxr
                pend_d[pl.ds(240, 16)] = idxd
                pltpu.async_copy(
                    g_hbm.at[plsc.Indices(pend_r.at[pl.ds(240, 16)],
                                          ignored_value=-1)],
                    rows0.at[pl.ds(0, 16)], gsem).wait()
                pltpu.sync_copy(
                    rows0.at[pl.ds(0, 16)],
                    acc.at[plsc.Indices(pend_d.at[pl.ds(240, 16)],
                                        ignored_value=-1)],
                    add=True)
                return 0

            lax.fori_loop(0, FB // 16, tail_body, 0)
            plsc.subcore_barrier()
            pltpu.sync_copy(acc.at[pl.ds(s * (CHUNK // NS), CHUNK // NS)],
                            out_hbm.at[pl.ds(base + s * (CHUNK // NS),
                                             CHUNK // NS)])
            plsc.subcore_barrier()

        return 0

    lax.fori_loop(0, CPC, chunk_body, 0)


def _sc_seg_sum_big(g, row, col, zeros16):
    return pl.kernel(
        _sc_big_body,
        out_type=jax.ShapeDtypeStruct((NPAD_BIG, D), jnp.float32),
        mesh=_mesh,
        compiler_params=pltpu.CompilerParams(needs_layout_passes=False,
                                             use_tc_tiling_on_sc=False),
        scratch_types=[
            pltpu.VMEM_SHARED((CHUNK, D), jnp.float32),
            pltpu.VMEM((BLK,), jnp.int32),
            pltpu.VMEM((BLK,), jnp.int32),
            pltpu.VMEM((256,), jnp.int32),
            pltpu.VMEM((256,), jnp.int32),
            pltpu.VMEM((FB,), jnp.int32),
            pltpu.VMEM((FB,), jnp.int32),
            pltpu.VMEM((FB,), jnp.int32),
            pltpu.VMEM((FB,), jnp.int32),
            pltpu.VMEM((FB, D), jnp.float32),
            pltpu.VMEM((FB, D), jnp.float32),
            pltpu.SemaphoreType.DMA,
        ],
    )(g, row, col, zeros16)


def _sc_small_body(g_hbm, row2_hbm, col2_hbm, zeros_hbm, out_hbm,
                   acc, rowb, colb, rows_v, zbuf, gsem):
    c = lax.axis_index("c")
    s = lax.axis_index("s")
    pltpu.sync_copy(zeros_hbm, zbuf)
    for z in range(RPT0 // ZR):
        pltpu.sync_copy(zbuf, acc.at[pl.ds(s * RPT0 + z * ZR, ZR)])
    plsc.subcore_barrier()
    roff = c * (EPC // SB) + s * NSB
    pltpu.sync_copy(row2_hbm.at[pl.ds(roff, NSB)], rowb)
    pltpu.sync_copy(col2_hbm.at[pl.ds(roff, NSB)], colb)

    def sb_body(b, _):
        pltpu.async_copy(g_hbm.at[rowb.at[b]], rows_v, gsem).wait()
        pltpu.sync_copy(rows_v, acc.at[colb.at[b]], add=True)
        return 0

    lax.fori_loop(0, NSB, sb_body, 0)
    plsc.subcore_barrier()
    pltpu.sync_copy(acc.at[pl.ds(s * RPT0, RPT0)],
                    out_hbm.at[c].at[pl.ds(s * RPT0, RPT0)])


def _sc_seg_sum_small(g, row2, col2, zeros_sm):
    return pl.kernel(
        _sc_small_body,
        out_type=jax.ShapeDtypeStruct((NC, N0PAD, D0), jnp.float32),
        mesh=_mesh,
        compiler_params=pltpu.CompilerParams(use_tc_tiling_on_sc=False),
        scratch_types=[
            pltpu.VMEM_SHARED((N0PAD, D0), jnp.float32),
            pltpu.VMEM((NSB, SB), jnp.int32),
            pltpu.VMEM((NSB, SB), jnp.int32),
            pltpu.VMEM((SB, D0), jnp.float32),
            pltpu.VMEM((ZR, D0), jnp.float32),
            pltpu.SemaphoreType.DMA,
        ],
    )(g, row2, col2, zeros_sm)


RB = 1000
RB0 = 1568


def _tc_dis_body(deg2_ref, q_ref, dis_ref, uq_ref):
    dsum = deg2_ref[0] + deg2_ref[1]
    dis = jnp.where(dsum > 0, lax.rsqrt(jnp.maximum(dsum, 1e-12)), 0.0)
    dis_ref[...] = dis
    uq_ref[...] = dis * q_ref[...]


def _tc_dis(deg2, qpad):
    return pl.pallas_call(
        _tc_dis_body,
        grid=(N0PAD // RB0,),
        in_specs=[
            pl.BlockSpec((NC, RB0, D0), lambda i: (0, i, 0)),
            pl.BlockSpec((RB0, D0), lambda i: (i, 0)),
        ],
        out_specs=[
            pl.BlockSpec((RB0, D0), lambda i: (i, 0)),
            pl.BlockSpec((RB0, D0), lambda i: (i, 0)),
        ],
        out_shape=[
            jax.ShapeDtypeStruct((N0PAD, D0), jnp.float32),
            jax.ShapeDtypeStruct((N0PAD, D0), jnp.float32),
        ],
    )(deg2, qpad)


def _tc_w_small_body(v2_ref, dis_ref, w_ref):
    d = dis_ref[...]
    w_ref[...] = d * d * (v2_ref[0] + v2_ref[1])


def _tc_w_small(v2, dis16):
    return pl.pallas_call(
        _tc_w_small_body,
        grid=(N0PAD // RB0,),
        in_specs=[
            pl.BlockSpec((NC, RB0, D0), lambda i: (0, i, 0)),
            pl.BlockSpec((RB0, D0), lambda i: (i, 0)),
        ],
        out_specs=pl.BlockSpec((RB0, D0), lambda i: (i, 0)),
        out_shape=jax.ShapeDtypeStruct((N0PAD, D0), jnp.float32),
    )(v2, dis16)


def _tc_stage0_body(q_ref, dis_ref, v1_ref, v2_ref, v3_ref, g_ref, b0_ref,
                    lng_ref, lnb_ref, x1_ref, u1_ref):
    dis8 = dis_ref[:, :8]
    coef = jnp.concatenate([
        q_ref[:, :8],
        dis8 * (v1_ref[0, :, :8] + v1_ref[1, :, :8]),
        dis8 * (v2_ref[0, :, :8] + v2_ref[1, :, :8]),
        dis8 * (v3_ref[0, :, :8] + v3_ref[1, :, :8]),
    ], axis=1)
    x = jnp.dot(coef, g_ref[...], preferred_element_type=jnp.float32,
                precision=lax.Precision.HIGHEST)
    x = x + b0_ref[...]
    x3 = x.reshape(RB, M, HID)
    mu = jnp.mean(x3, axis=2, keepdims=True)
    xc = x3 - mu
    var = jnp.mean(xc * xc, axis=2, keepdims=True)
    xn = xc * lax.rsqrt(var + 1e-5)
    xn = xn.reshape(RB, D) * lng_ref[...] + lnb_ref[...]
    x1 = jnp.maximum(xn, 0.0)
    x1_ref[...] = x1
    u1_ref[...] = dis_ref[:, :1] * x1


def _tc_stage0(qpad, dis16, vq1, vq2, vq3, G, b0t, lngt, lnbt):
    return pl.pallas_call(
        _tc_stage0_body,
        grid=(N // RB,),
        in_specs=[
            pl.BlockSpec((RB, D0), lambda i: (i, 0)),
            pl.BlockSpec((RB, D0), lambda i: (i, 0)),
            pl.BlockSpec((NC, RB, D0), lambda i: (0, i, 0)),
            pl.BlockSpec((NC, RB, D0), lambda i: (0, i, 0)),
            pl.BlockSpec((NC, RB, D0), lambda i: (0, i, 0)),
            pl.BlockSpec((32, D), lambda i: (0, 0)),
            pl.BlockSpec((1, D), lambda i: (0, 0)),
            pl.BlockSpec((1, D), lambda i: (0, 0)),
            pl.BlockSpec((1, D), lambda i: (0, 0)),
        ],
        out_specs=[
            pl.BlockSpec((RB, D), lambda i: (i, 0)),
            pl.BlockSpec((RB, D), lambda i: (i, 0)),
        ],
        out_shape=[
            jax.ShapeDtypeStruct((N, D), jnp.float32),
            jax.ShapeDtypeStruct((N, D), jnp.float32),
        ],
    )(qpad, dis16, vq1, vq2, vq3, G, b0t, lngt, lnbt)


def _tc_w_big_body(v_ref, dis_ref, w_ref):
    d = dis_ref[:, :1]
    w_ref[...] = d * d * v_ref[...]


def _tc_w_big(v, dis16):
    return pl.pallas_call(
        _tc_w_big_body,
        grid=(N // RB,),
        in_specs=[
            pl.BlockSpec((RB, D), lambda i: (i, 0)),
            pl.BlockSpec((RB, D0), lambda i: (i, 0)),
        ],
        out_specs=pl.BlockSpec((RB, D), lambda i: (i, 0)),
        out_shape=jax.ShapeDtypeStruct((N, D), jnp.float32),
    )(v, dis16)


def _tc_layer1_body(x_ref, v1_ref, v2_ref, v3_ref, dis_ref, bd_ref, b_ref,
                    x2_ref, u2_ref):
    x = x_ref[...]
    dis = dis_ref[:, :1]
    acc = jnp.dot(x, bd_ref[0], preferred_element_type=jnp.float32,
                precision=lax.Precision.HIGHEST)
    s = jnp.dot(v1_ref[...], bd_ref[1], preferred_element_type=jnp.float32,
                precision=lax.Precision.HIGHEST)
    s = s + jnp.dot(v2_ref[...], bd_ref[2], preferred_element_type=jnp.float32,
                precision=lax.Precision.HIGHEST)
    s = s + jnp.dot(v3_ref[...], bd_ref[3], preferred_element_type=jnp.float32,
                precision=lax.Precision.HIGHEST)
    acc = acc + dis * s + b_ref[...]
    x2 = jnp.maximum(acc, 0.0) + x
    x2_ref[...] = x2
    u2_ref[...] = dis * x2


def _tc_layer1(x1, v1, v2, v3, dis16, bd, bt):
    return pl.pallas_call(
        _tc_layer1_body,
        grid=(N // RB,),
        in_specs=[
            pl.BlockSpec((RB, D), lambda i: (i, 0)),
            pl.BlockSpec((RB, D), lambda i: (i, 0)),
            pl.BlockSpec((RB, D), lambda i: (i, 0)),
            pl.BlockSpec((RB, D), lambda i: (i, 0)),
            pl.BlockSpec((RB, D0), lambda i: (i, 0)),
            pl.BlockSpec((4, D, D), lambda i: (0, 0, 0)),
            pl.BlockSpec((1, D), lambda i: (0, 0)),
        ],
        out_specs=[
            pl.BlockSpec((RB, D), lambda i: (i, 0)),
            pl.BlockSpec((RB, D), lambda i: (i, 0)),
        ],
        out_shape=[
            jax.ShapeDtypeStruct((N, D), jnp.float32),
            jax.ShapeDtypeStruct((N, D), jnp.float32),
        ],
    )(x1, v1, v2, v3, dis16, bd, bt)


def _tc_layer2_body(x_ref, v1_ref, v2_ref, v3_ref, dis_ref, bd_ref, b_ref,
                    pt_ref, pb_ref, pooled_ref):
    x = x_ref[...]
    dis = dis_ref[:, :1]
    acc = jnp.dot(x, bd_ref[0], preferred_element_type=jnp.float32,
                precision=lax.Precision.HIGHEST)
    s = jnp.dot(v1_ref[...], bd_ref[1], preferred_element_type=jnp.float32,
                precision=lax.Precision.HIGHEST)
    s = s + jnp.dot(v2_ref[...], bd_ref[2], preferred_element_type=jnp.float32,
                precision=lax.Precision.HIGHEST)
    s = s + jnp.dot(v3_ref[...], bd_ref[3], preferred_element_type=jnp.float32,
                precision=lax.Precision.HIGHEST)
    h3 = acc + dis * s + b_ref[...]
    pooled_ref[...] = (
        jnp.dot(h3, pt_ref[...], preferred_element_type=jnp.float32,
                precision=lax.Precision.HIGHEST)
        + pb_ref[...])


def _tc_layer2(x2, v1, v2, v3, dis16, bd, bt, pt, pjb):
    return pl.pallas_call(
        _tc_layer2_body,
        grid=(N // RB,),
        in_specs=[
            pl.BlockSpec((RB, D), lambda i: (i, 0)),
            pl.BlockSpec((RB, D), lambda i: (i, 0)),
            pl.BlockSpec((RB, D), lambda i: (i, 0)),
            pl.BlockSpec((RB, D), lambda i: (i, 0)),
            pl.BlockSpec((RB, D0), lambda i: (i, 0)),
            pl.BlockSpec((4, D, D), lambda i: (0, 0, 0)),
            pl.BlockSpec((1, D), lambda i: (0, 0)),
            pl.BlockSpec((D, DM), lambda i: (0, 0)),
            pl.BlockSpec((1, DM), lambda i: (0, 0)),
        ],
        out_specs=pl.BlockSpec((RB, DM), lambda i: (i, 0)),
        out_shape=jax.ShapeDtypeStruct((N, DM), jnp.float32),
    )(x2, v1, v2, v3, dis16, bd, bt, pt, pjb)


def _tc_bn_sum_body(p_ref, s_ref):
    @pl.when(pl.program_id(0) == 0)
    def _():
        s_ref[...] = jnp.zeros_like(s_ref)

    s_ref[...] += jnp.sum(p_ref[...], axis=0, keepdims=True)


def _tc_bn_sum(pooled):
    return pl.pallas_call(
        _tc_bn_sum_body,
        grid=(N // RB,),
        in_specs=[pl.BlockSpec((RB, DM), lambda i: (i, 0))],
        out_specs=pl.BlockSpec((1, DM), lambda i: (0, 0)),
        out_shape=jax.ShapeDtypeStruct((1, DM), jnp.float32),
    )(pooled)


def _tc_bn_var_body(p_ref, mu_ref, s_ref):
    @pl.when(pl.program_id(0) == 0)
    def _():
        s_ref[...] = jnp.zeros_like(s_ref)

    d = p_ref[...] - mu_ref[...]
    s_ref[...] += jnp.sum(d * d, axis=0, keepdims=True)


def _tc_bn_var(pooled, mu):
    return pl.pallas_call(
        _tc_bn_var_body,
        grid=(N // RB,),
        in_specs=[
            pl.BlockSpec((RB, DM), lambda i: (i, 0)),
            pl.BlockSpec((1, DM), lambda i: (0, 0)),
        ],
        out_specs=pl.BlockSpec((1, DM), lambda i: (0, 0)),
        out_shape=jax.ShapeDtypeStruct((1, DM), jnp.float32),
    )(pooled, mu)


def _tc_bn_norm_body(p_ref, mu_ref, rstd_ref, g_ref, b_ref, out_ref):
    out_ref[...] = ((p_ref[...] - mu_ref[...]) * rstd_ref[...] * g_ref[...]
                    + b_ref[...])


def _tc_bn_norm(pooled, mu, rstd, g, b):
    return pl.pallas_call(
        _tc_bn_norm_body,
        grid=(N // RB,),
        in_specs=[
            pl.BlockSpec((RB, DM), lambda i: (i, 0)),
            pl.BlockSpec((1, DM), lambda i: (0, 0)),
            pl.BlockSpec((1, DM), lambda i: (0, 0)),
            pl.BlockSpec((1, DM), lambda i: (0, 0)),
            pl.BlockSpec((1, DM), lambda i: (0, 0)),
        ],
        out_specs=pl.BlockSpec((RB, DM), lambda i: (i, 0)),
        out_shape=jax.ShapeDtypeStruct((N, DM), jnp.float32),
    )(pooled, mu, rstd, g, b)


def kernel(x, edge_index, Q, conv0_W, conv0_b, conv1_W, conv1_b, conv2_W,
           conv2_b, ln_g, ln_b, proj_W, proj_b, bn_g, bn_b):
    row = edge_index[0]
    col = edge_index[1]

    pad = EPAD - E
    row2 = jnp.concatenate(
        [row, jnp.zeros((pad,), jnp.int32)]).reshape(EPAD // SB, SB)
    col2 = jnp.concatenate(
        [col, jnp.full((pad,), N, jnp.int32)]).reshape(EPAD // SB, SB)
    qpad = jnp.zeros((N0PAD, D0), jnp.float32).at[:N, :M].set(Q)
    ones_tbl = jnp.ones((N0PAD, D0), jnp.float32)
    zeros_sm = jnp.zeros((ZR, D0), jnp.float32)
    zeros16 = jnp.zeros((16, D), jnp.float32)

    eye8 = jnp.eye(M, dtype=jnp.float32)
    G = (eye8[None, :, :, None] * conv0_W[:, 0][:, None, None, :]).reshape(
        4 * M, D)
    b0t = jnp.tile(conv0_b, M)[None]
    lngt = jnp.tile(ln_g, M)[None]
    lnbt = jnp.tile(ln_b, M)[None]
    bd1 = jnp.stack([jnp.kron(eye8, conv1_W[k]) for k in range(4)])
    b1t = jnp.tile(conv1_b, M)[None]
    bd2 = jnp.stack([jnp.kron(eye8, conv2_W[k]) for k in range(4)])
    b2t = jnp.tile(conv2_b, M)[None]
    pt = jnp.tile(proj_W, (M, 1)) / M
    pjb = proj_b[None]

    deg2 = _sc_seg_sum_small(ones_tbl, row2, col2, zeros_sm)
    dis16, uq = _tc_dis(deg2, qpad)

    vq1 = _sc_seg_sum_small(uq, row2, col2, zeros_sm)
    wq1 = _tc_w_small(vq1, dis16)
    vq2 = _sc_seg_sum_small(wq1, row2, col2, zeros_sm)
    wq2 = _tc_w_small(vq2, dis16)
    vq3 = _sc_seg_sum_small(wq2, row2, col2, zeros_sm)
    x1, u1 = _tc_stage0(qpad, dis16, vq1, vq2, vq3, G, b0t, lngt, lnbt)

    v11 = _sc_seg_sum_big(u1, row, col, zeros16)
    w11 = _tc_w_big(v11, dis16)
    v12 = _sc_seg_sum_big(w11, row, col, zeros16)
    w12 = _tc_w_big(v12, dis16)
    v13 = _sc_seg_sum_big(w12, row, col, zeros16)
    x2, u2 = _tc_layer1(x1, v11, v12, v13, dis16, bd1, b1t)

    v21 = _sc_seg_sum_big(u2, row, col, zeros16)
    w21 = _tc_w_big(v21, dis16)
    v22 = _sc_seg_sum_big(w21, row, col, zeros16)
    w22 = _tc_w_big(v22, dis16)
    v23 = _sc_seg_sum_big(w22, row, col, zeros16)
    pooled = _tc_layer2(x2, v21, v22, v23, dis16, bd2, b2t, pt, pjb)

    mu = _tc_bn_sum(pooled) / N
    var = _tc_bn_var(pooled, mu) / N
    rstd = lax.rsqrt(var + 1e-5)
    return _tc_bn_norm(pooled, mu, rstd, bn_g[None], bn_b[None])
